# Initial kernel scaffold; baseline (speedup 1.0000x reference)
import functools

import jax
import jax.numpy as jnp
from jax import lax
from jax.experimental import pallas as pl
from jax.experimental.pallas import tpu as pltpu
from jax.experimental.pallas import tpu_sc as plsc

NUM_EMBEDDINGS = 1000000
EMBEDDING_DIM = 32
BATCH = 16384
FIELDS = 26

_B = BATCH * FIELDS
_NW = 32
_CHUNK = 128
_CHUNKS_PER_W = _B // (_NW * _CHUNK)
_NBUF = 2


@functools.partial(
    pl.kernel,
    out_type=jax.ShapeDtypeStruct((_B, EMBEDDING_DIM), jnp.float32),
    mesh=plsc.VectorSubcoreMesh(core_axis_name="c", subcore_axis_name="s"),
    scratch_types=[
        pltpu.VMEM((_CHUNKS_PER_W, _CHUNK), jnp.int32),
        pltpu.VMEM((_NBUF, _CHUNK, EMBEDDING_DIM), jnp.float32),
        pltpu.SemaphoreType.DMA,
        pltpu.SemaphoreType.DMA,
    ],
)
def _gather_kernel(idx_hbm, table_hbm, out_hbm, idx_v, rows_v, sem0, sem1):
    wid = lax.axis_index("s") * 2 + lax.axis_index("c")
    chunk_base = wid * _CHUNKS_PER_W
    pltpu.sync_copy(idx_hbm.at[pl.ds(chunk_base, _CHUNKS_PER_W)], idx_v)

    sems = (sem0, sem1)

    def gather(j, b):
        return pltpu.make_async_copy(
            table_hbm.at[idx_v.at[j]], rows_v.at[b], sems[b]
        )

    for b in range(_NBUF):
        gather(b, b).start()

    def body(g, _):
        for b in range(_NBUF):
            j = g * _NBUF + b
            gather(j, b).wait()
            row0 = (chunk_base + j) * _CHUNK
            pltpu.sync_copy(rows_v.at[b], out_hbm.at[pl.ds(row0, _CHUNK)])

            @pl.when(j + _NBUF < _CHUNKS_PER_W)
            def _():
                gather(j + _NBUF, b).start()

        return 0

    lax.fori_loop(0, _CHUNKS_PER_W // _NBUF, body, 0)


def kernel(x, weight):
    idx = x.reshape(_NW * _CHUNKS_PER_W, _CHUNK).astype(jnp.int32)
    out = _gather_kernel(idx, weight)
    return out.reshape(BATCH, FIELDS, EMBEDDING_DIM)


# speedup vs baseline: 1.5224x; 1.5224x over previous
import functools

import jax
import jax.numpy as jnp
from jax import lax
from jax.experimental import pallas as pl
from jax.experimental.pallas import tpu as pltpu
from jax.experimental.pallas import tpu_sc as plsc

NUM_EMBEDDINGS = 1000000
EMBEDDING_DIM = 32
BATCH = 16384
FIELDS = 26

_B = BATCH * FIELDS
_NW = 32
_CHUNK = 128
_CHUNKS_PER_W = _B // (_NW * _CHUNK)
_NBUF = 2


@functools.partial(
    pl.kernel,
    out_type=jax.ShapeDtypeStruct((_B, EMBEDDING_DIM), jnp.float32),
    mesh=plsc.VectorSubcoreMesh(core_axis_name="c", subcore_axis_name="s"),
    scratch_types=[
        pltpu.VMEM((_CHUNKS_PER_W, _CHUNK), jnp.int32),
        pltpu.VMEM((_NBUF, _CHUNK, EMBEDDING_DIM), jnp.float32),
        pltpu.SemaphoreType.DMA,
        pltpu.SemaphoreType.DMA,
    ],
    compiler_params=pltpu.CompilerParams(use_tc_tiling_on_sc=False),
)
def _gather_kernel(idx_hbm, table_hbm, out_hbm, idx_v, rows_v, sem0, sem1):
    wid = lax.axis_index("s") * 2 + lax.axis_index("c")
    chunk_base = wid * _CHUNKS_PER_W
    pltpu.sync_copy(idx_hbm.at[pl.ds(chunk_base, _CHUNKS_PER_W)], idx_v)

    sems = (sem0, sem1)

    def gather(j, b):
        return pltpu.make_async_copy(
            table_hbm.at[idx_v.at[j]], rows_v.at[b], sems[b]
        )

    for b in range(_NBUF):
        gather(b, b).start()

    def body(g, _):
        for b in range(_NBUF):
            j = g * _NBUF + b
            gather(j, b).wait()
            row0 = (chunk_base + j) * _CHUNK
            pltpu.sync_copy(rows_v.at[b], out_hbm.at[pl.ds(row0, _CHUNK)])

            @pl.when(j + _NBUF < _CHUNKS_PER_W)
            def _():
                gather(j + _NBUF, b).start()

        return 0

    lax.fori_loop(0, _CHUNKS_PER_W // _NBUF, body, 0)


def kernel(x, weight):
    idx = x.reshape(_NW * _CHUNKS_PER_W, _CHUNK).astype(jnp.int32)
    out = _gather_kernel(idx, weight)
    return out.reshape(BATCH, FIELDS, EMBEDDING_DIM)


# speedup vs baseline: 1.5740x; 1.0339x over previous
import functools

import jax
import jax.numpy as jnp
from jax import lax
from jax.experimental import pallas as pl
from jax.experimental.pallas import tpu as pltpu
from jax.experimental.pallas import tpu_sc as plsc

NUM_EMBEDDINGS = 1000000
EMBEDDING_DIM = 32
BATCH = 16384
FIELDS = 26

_B = BATCH * FIELDS
_NW = 32
_CHUNK = 128
_CHUNKS_PER_W = _B // (_NW * _CHUNK)
_NBUF = 8


@functools.partial(
    pl.kernel,
    out_type=jax.ShapeDtypeStruct((_B, EMBEDDING_DIM), jnp.float32),
    mesh=plsc.VectorSubcoreMesh(core_axis_name="c", subcore_axis_name="s"),
    scratch_types=[
        pltpu.VMEM((_CHUNKS_PER_W, _CHUNK), jnp.int32),
        pltpu.VMEM((_NBUF, _CHUNK, EMBEDDING_DIM), jnp.float32),
        [pltpu.SemaphoreType.DMA] * _NBUF,
        [pltpu.SemaphoreType.DMA] * _NBUF,
    ],
    compiler_params=pltpu.CompilerParams(use_tc_tiling_on_sc=False),
)
def _gather_kernel(idx_hbm, table_hbm, out_hbm, idx_v, rows_v, gsems, wsems):
    wid = lax.axis_index("s") * 2 + lax.axis_index("c")
    chunk_base = wid * _CHUNKS_PER_W
    pltpu.sync_copy(idx_hbm.at[pl.ds(chunk_base, _CHUNKS_PER_W)], idx_v)

    def gather(j, b):
        return pltpu.make_async_copy(
            table_hbm.at[idx_v.at[j]], rows_v.at[b], gsems[b]
        )

    def writeback(j, b):
        row0 = (chunk_base + j) * _CHUNK
        return pltpu.make_async_copy(
            rows_v.at[b], out_hbm.at[pl.ds(row0, _CHUNK)], wsems[b]
        )

    for b in range(_NBUF):
        gather(b, b).start()

    def body(g, _):
        for b in range(_NBUF):
            j = g * _NBUF + b
            gather(j, b).wait()
            writeback(j, b).start()
        for b in range(_NBUF):
            j = g * _NBUF + b
            writeback(j, b).wait()

            @pl.when(j + _NBUF < _CHUNKS_PER_W)
            def _():
                gather(j + _NBUF, b).start()

        return 0

    lax.fori_loop(0, _CHUNKS_PER_W // _NBUF, body, 0)


def kernel(x, weight):
    idx = x.reshape(_NW * _CHUNKS_PER_W, _CHUNK).astype(jnp.int32)
    out = _gather_kernel(idx, weight)
    return out.reshape(BATCH, FIELDS, EMBEDDING_DIM)


# speedup vs baseline: 1.5852x; 1.0071x over previous
import functools

import jax
import jax.numpy as jnp
from jax import lax
from jax.experimental import pallas as pl
from jax.experimental.pallas import tpu as pltpu
from jax.experimental.pallas import tpu_sc as plsc

NUM_EMBEDDINGS = 1000000
EMBEDDING_DIM = 32
BATCH = 16384
FIELDS = 26

_B = BATCH * FIELDS
_NW = 32
_CHUNK = 128
_CHUNKS_PER_W = _B // (_NW * _CHUNK)
_NBUF = 8


@functools.partial(
    pl.kernel,
    out_type=jax.ShapeDtypeStruct((_B, EMBEDDING_DIM), jnp.float32),
    mesh=plsc.VectorSubcoreMesh(core_axis_name="c", subcore_axis_name="s"),
    scratch_types=[
        pltpu.VMEM((_CHUNKS_PER_W, _CHUNK), jnp.int32),
        pltpu.VMEM((_NBUF, _CHUNK, EMBEDDING_DIM), jnp.float32),
        [pltpu.SemaphoreType.DMA] * _NBUF,
        [pltpu.SemaphoreType.DMA] * _NBUF,
    ],
    compiler_params=pltpu.CompilerParams(use_tc_tiling_on_sc=False),
)
def _gather_kernel(idx_hbm, table_hbm, out_hbm, idx_v, rows_v, gsems, wsems):
    wid = lax.axis_index("s") * 2 + lax.axis_index("c")
    chunk_base = wid * _CHUNKS_PER_W
    pltpu.sync_copy(idx_hbm.at[pl.ds(chunk_base, _CHUNKS_PER_W)], idx_v)

    def gather(j, b):
        return pltpu.make_async_copy(
            table_hbm.at[idx_v.at[j]], rows_v.at[b], gsems[b]
        )

    def writeback(j, b):
        row0 = (chunk_base + j) * _CHUNK
        return pltpu.make_async_copy(
            rows_v.at[b], out_hbm.at[pl.ds(row0, _CHUNK)], wsems[b]
        )

    for b in range(_NBUF):
        gather(b, b).start()

    def body(g, _):
        for b in range(_NBUF):
            j = g * _NBUF + b
            gather(j, b).wait()
            writeback(j, b).start()
        for b in range(_NBUF):
            j = g * _NBUF + b
            writeback(j, b).wait()

            @pl.when(j + _NBUF < _CHUNKS_PER_W)
            def _():
                gather(j + _NBUF, b).start()

        return 0

    lax.fori_loop(0, _CHUNKS_PER_W // _NBUF, body, 0)


_TCOLS = 2048
_TROWS = _TCOLS * EMBEDDING_DIM // 128
_TGRID = (NUM_EMBEDDINGS + _TCOLS - 1) // _TCOLS
_TPAD_ROWS = _TGRID * _TCOLS


def _linearize_body(wt_ref, o_ref):
    xt = wt_ref[...].T
    xt4 = xt.reshape(_TROWS, 4, EMBEDDING_DIM)
    for a in range(4):
        o_ref[0, :, 32 * a:32 * (a + 1)] = xt4[:, a, :]


def _linearize(wt):
    return pl.pallas_call(
        _linearize_body,
        grid=(_TGRID,),
        in_specs=[pl.BlockSpec((EMBEDDING_DIM, _TCOLS), lambda i: (0, i))],
        out_specs=pl.BlockSpec((1, _TROWS, 128), lambda i: (i, 0, 0)),
        out_shape=jax.ShapeDtypeStruct((_TGRID, _TROWS, 128), jnp.float32),
    )(wt)


def kernel(x, weight):
    idx = x.reshape(_NW * _CHUNKS_PER_W, _CHUNK).astype(jnp.int32)
    lin3 = _linearize(weight.T)
    table = lin3.reshape(_TPAD_ROWS, EMBEDDING_DIM)
    out = _gather_kernel(idx, table)
    return out.reshape(BATCH, FIELDS, EMBEDDING_DIM)


# speedup vs baseline: 1.8908x; 1.1927x over previous
import functools

import jax
import jax.numpy as jnp
from jax import lax
from jax.experimental import pallas as pl
from jax.experimental.pallas import tpu as pltpu
from jax.experimental.pallas import tpu_sc as plsc

NUM_EMBEDDINGS = 1000000
EMBEDDING_DIM = 32
BATCH = 16384
FIELDS = 26

_B = BATCH * FIELDS
_NW = 32
_CHUNK = 128
_CHUNKS_PER_W = _B // (_NW * _CHUNK)
_NBUF = 8


@functools.partial(
    pl.kernel,
    out_type=jax.ShapeDtypeStruct((_B, EMBEDDING_DIM), jnp.float32),
    mesh=plsc.VectorSubcoreMesh(core_axis_name="c", subcore_axis_name="s"),
    scratch_types=[
        pltpu.VMEM((_CHUNKS_PER_W, _CHUNK), jnp.int32),
        pltpu.VMEM((_NBUF, _CHUNK, EMBEDDING_DIM), jnp.float32),
        [pltpu.SemaphoreType.DMA] * _NBUF,
        [pltpu.SemaphoreType.DMA] * _NBUF,
    ],
    compiler_params=pltpu.CompilerParams(use_tc_tiling_on_sc=False),
)
def _gather_kernel(idx_hbm, table_hbm, out_hbm, idx_v, rows_v, gsems, wsems):
    wid = lax.axis_index("s") * 2 + lax.axis_index("c")
    chunk_base = wid * _CHUNKS_PER_W
    pltpu.sync_copy(idx_hbm.at[pl.ds(chunk_base, _CHUNKS_PER_W)], idx_v)

    def gather(j, b):
        return pltpu.make_async_copy(
            table_hbm.at[idx_v.at[j]], rows_v.at[b], gsems[b]
        )

    def writeback(j, b):
        row0 = (chunk_base + j) * _CHUNK
        return pltpu.make_async_copy(
            rows_v.at[b], out_hbm.at[pl.ds(row0, _CHUNK)], wsems[b]
        )

    for b in range(_NBUF):
        gather(b, b).start()

    def body(g, _):
        for b in range(_NBUF):
            j = g * _NBUF + b
            gather(j, b).wait()
            writeback(j, b).start()
        for b in range(_NBUF):
            j = g * _NBUF + b
            writeback(j, b).wait()

            @pl.when(j + _NBUF < _CHUNKS_PER_W)
            def _():
                gather(j + _NBUF, b).start()

        return 0

    lax.fori_loop(0, _CHUNKS_PER_W // _NBUF, body, 0)


_TCOLS = 16384
_TROWS = _TCOLS * EMBEDDING_DIM // 128
_TGRID = (NUM_EMBEDDINGS + _TCOLS - 1) // _TCOLS
_TPAD_ROWS = _TGRID * _TCOLS


def _linearize_body(wt_ref, o_ref):
    xt = wt_ref[...].T
    xt4 = xt.reshape(_TROWS, 4, EMBEDDING_DIM)
    for a in range(4):
        o_ref[0, :, 32 * a:32 * (a + 1)] = xt4[:, a, :]


def _linearize(wt):
    return pl.pallas_call(
        _linearize_body,
        grid=(_TGRID,),
        in_specs=[pl.BlockSpec((EMBEDDING_DIM, _TCOLS), lambda i: (0, i))],
        out_specs=pl.BlockSpec((1, _TROWS, 128), lambda i: (i, 0, 0)),
        out_shape=jax.ShapeDtypeStruct((_TGRID, _TROWS, 128), jnp.float32),
    )(wt)


def kernel(x, weight):
    idx = x.reshape(_NW * _CHUNKS_PER_W, _CHUNK).astype(jnp.int32)
    lin3 = _linearize(weight.T)
    table = lin3.reshape(_TPAD_ROWS, EMBEDDING_DIM)
    out = _gather_kernel(idx, table)
    return out.reshape(BATCH, FIELDS, EMBEDDING_DIM)
